# baseline (speedup 1.0000x reference)
import jax
import jax.numpy as jnp
from jax.experimental import pallas as pl
from jax.experimental.pallas import tpu as pltpu


def _se_body(x_ref, w1_ref, w2_ref, out_ref):
    x = x_ref[...]
    pooled = jnp.sum(x, axis=-1, dtype=jnp.float32)
    z = jnp.maximum(
        jnp.dot(pooled, w1_ref[...], preferred_element_type=jnp.float32), 0.0)
    gate = jax.nn.sigmoid(
        jnp.dot(z, w2_ref[...], preferred_element_type=jnp.float32))
    out_ref[...] = x * gate.astype(x.dtype)[:, :, None]


def _pick_block_n(n, plane_bytes, budget_bytes):
    bn = max(1, min(n, budget_bytes // max(1, plane_bytes)))
    while n % bn:
        bn -= 1
    return bn


def kernel(x_nchw, se_fc1_w, se_fc2_w):
    N, C, H, W = x_nchw.shape
    HW = H * W
    hidden = se_fc1_w.shape[0]
    dtype = x_nchw.dtype

    x3 = x_nchw.reshape(N, C, HW)
    w1 = (jnp.transpose(se_fc1_w) * (1.0 / HW)).astype(jnp.float32)
    w2 = jnp.transpose(se_fc2_w).astype(jnp.float32)

    plane_bytes = C * HW * dtype.itemsize
    bn = _pick_block_n(N, plane_bytes, 4 << 20)
    num_blocks = N // bn
    block_bytes = bn * plane_bytes
    fixed_bytes = 2 * C * hidden * 4 + (6 << 20)
    vmem_limit = int(min(64 << 20, max(32 << 20, 4 * block_bytes + fixed_bytes)))

    out3 = pl.pallas_call(
        _se_body,
        out_shape=jax.ShapeDtypeStruct((N, C, HW), dtype),
        grid=(num_blocks,),
        in_specs=[
            pl.BlockSpec((bn, C, HW), lambda n: (n, 0, 0)),
            pl.BlockSpec((C, hidden), lambda n: (0, 0)),
            pl.BlockSpec((hidden, C), lambda n: (0, 0)),
        ],
        out_specs=pl.BlockSpec((bn, C, HW), lambda n: (n, 0, 0)),
        compiler_params=pltpu.CompilerParams(
            dimension_semantics=("parallel",),
            vmem_limit_bytes=vmem_limit),
    )(x3, w1, w2)
    return out3.reshape(N, C, H, W)


# speedup vs baseline: 1.0086x; 1.0086x over previous
import jax
import jax.numpy as jnp
from jax.experimental import pallas as pl
from jax.experimental.pallas import tpu as pltpu


def _se_body(x_ref, w1_ref, w2_ref, out_ref):
    x = x_ref[...]
    pooled = jnp.sum(x, axis=-1, dtype=jnp.float32)
    z = jnp.maximum(
        jnp.dot(pooled, w1_ref[...], preferred_element_type=jnp.float32), 0.0)
    gate = jax.nn.sigmoid(
        jnp.dot(z, w2_ref[...], preferred_element_type=jnp.float32))
    out_ref[...] = x * gate.astype(x.dtype)[:, :, None]


def _pick_block_n(n, plane_bytes, budget_bytes):
    bn = max(1, min(n, budget_bytes // max(1, plane_bytes)))
    while n % bn:
        bn -= 1
    return bn


def kernel(x_nchw, se_fc1_w, se_fc2_w):
    N, C, H, W = x_nchw.shape
    HW = H * W
    hidden = se_fc1_w.shape[0]
    dtype = x_nchw.dtype

    x3 = x_nchw.reshape(N, C, HW)
    w1 = (jnp.transpose(se_fc1_w) * (1.0 / HW)).astype(jnp.float32)
    w2 = jnp.transpose(se_fc2_w).astype(jnp.float32)

    plane_bytes = C * HW * dtype.itemsize
    bn = _pick_block_n(N, plane_bytes, 16 << 20)
    num_blocks = N // bn
    block_bytes = bn * plane_bytes
    fixed_bytes = 2 * C * hidden * 4 + (6 << 20)
    vmem_limit = int(min(64 << 20, max(32 << 20, 4 * block_bytes + fixed_bytes)))

    out3 = pl.pallas_call(
        _se_body,
        out_shape=jax.ShapeDtypeStruct((N, C, HW), dtype),
        grid=(num_blocks,),
        in_specs=[
            pl.BlockSpec((bn, C, HW), lambda n: (n, 0, 0)),
            pl.BlockSpec((C, hidden), lambda n: (0, 0)),
            pl.BlockSpec((hidden, C), lambda n: (0, 0)),
        ],
        out_specs=pl.BlockSpec((bn, C, HW), lambda n: (n, 0, 0)),
        compiler_params=pltpu.CompilerParams(
            dimension_semantics=("parallel",),
            vmem_limit_bytes=vmem_limit),
    )(x3, w1, w2)
    return out3.reshape(N, C, H, W)


# speedup vs baseline: 1.0088x; 1.0002x over previous
import jax
import jax.numpy as jnp
from jax.experimental import pallas as pl
from jax.experimental.pallas import tpu as pltpu


def _se_body(x_ref, w1_ref, w2_ref, out_ref):
    x = x_ref[...]
    pooled = jnp.sum(x, axis=-1, dtype=jnp.float32)
    z = jnp.maximum(
        jnp.dot(pooled, w1_ref[...], preferred_element_type=jnp.float32), 0.0)
    gate = jax.nn.sigmoid(
        jnp.dot(z, w2_ref[...], preferred_element_type=jnp.float32))
    out_ref[...] = x * gate.astype(x.dtype)[:, :, None]


def _pick_block_n(n, plane_bytes, budget_bytes):
    bn = max(1, min(n, budget_bytes // max(1, plane_bytes)))
    while n % bn:
        bn -= 1
    return bn


def kernel(x_nchw, se_fc1_w, se_fc2_w):
    N, C, H, W = x_nchw.shape
    HW = H * W
    hidden = se_fc1_w.shape[0]
    dtype = x_nchw.dtype

    x3 = x_nchw.reshape(N, C, HW)
    w1 = (jnp.transpose(se_fc1_w) * (1.0 / HW)).astype(jnp.float32)
    w2 = jnp.transpose(se_fc2_w).astype(jnp.float32)

    plane_bytes = C * HW * dtype.itemsize
    bn = _pick_block_n(N, plane_bytes, 14 << 20)
    num_blocks = N // bn
    block_bytes = bn * plane_bytes
    fixed_bytes = 2 * C * hidden * 4 + (6 << 20)
    vmem_limit = int(min(64 << 20, max(32 << 20, 4 * block_bytes + fixed_bytes)))

    out3 = pl.pallas_call(
        _se_body,
        out_shape=jax.ShapeDtypeStruct((N, C, HW), dtype),
        grid=(num_blocks,),
        in_specs=[
            pl.BlockSpec((bn, C, HW), lambda n: (n, 0, 0)),
            pl.BlockSpec((C, hidden), lambda n: (0, 0)),
            pl.BlockSpec((hidden, C), lambda n: (0, 0)),
        ],
        out_specs=pl.BlockSpec((bn, C, HW), lambda n: (n, 0, 0)),
        compiler_params=pltpu.CompilerParams(
            dimension_semantics=("parallel",),
            vmem_limit_bytes=vmem_limit),
    )(x3, w1, w2)
    return out3.reshape(N, C, H, W)
